# baseline (speedup 1.0000x reference)
import functools

import jax
import jax.numpy as jnp
import numpy as np
from jax import lax
from jax.experimental import pallas as pl
from jax.experimental.pallas import tpu as pltpu
from jax.experimental.pallas import tpu_sc as plsc

F32 = jnp.float32
_BN_S = np.float32(1.0 / np.sqrt(1.0 + 1e-5))
_INV_SQRT2 = np.float32(1.0 / np.sqrt(2.0))
_PREC = lax.Precision.HIGHEST
_INTERP = False

N_NODES = 3136
B = 2
C = 192
KNN = 9
TM = 784


def _gelu(x):
    return 0.5 * x * (1.0 + lax.erf(x * _INV_SQRT2))


def _dot(a, b):
    return jax.lax.dot_general(a, b, (((1,), (0,)), ((), ())),
                               precision=_PREC, preferred_element_type=F32)


def _cmm(p3, warr, bias, act, nk, nn, c_out, pos=None, transpose_out=False):
    kc = p3.shape[1] // nk

    def body(*refs):
        if pos is not None:
            p_ref, w_ref, b_ref, pos_ref, o_ref, acc_ref = refs
        else:
            p_ref, w_ref, b_ref, o_ref, acc_ref = refs
        k = pl.program_id(1)
        z = _dot(w_ref[0], p_ref[0])

        @pl.when(k == 0)
        def _():
            acc_ref[...] = z

        @pl.when(k > 0)
        def _():
            acc_ref[...] += z

        @pl.when(k == nk - 1)
        def _():
            r = acc_ref[...] + b_ref[...]
            if act:
                r = _gelu(r)
            if pos is not None:
                r = r + pos_ref[...]
            if transpose_out:
                o_ref[...] = r.T
            else:
                o_ref[0] = r

    in_specs = [
        pl.BlockSpec((1, kc, nn), lambda b, k: (b, k, 0)),
        pl.BlockSpec((1, c_out, kc), lambda b, k: (k, 0, 0)),
        pl.BlockSpec((c_out, 1), lambda b, k: (0, 0)),
    ]
    args = [p3, warr, bias.reshape(c_out, 1)]
    if pos is not None:
        in_specs.append(pl.BlockSpec((c_out, nn), lambda b, k: (0, 0)))
        args.append(pos)
    if transpose_out:
        out_specs = pl.BlockSpec((nn, c_out), lambda b, k: (b, 0))
        out_shape = jax.ShapeDtypeStruct((B * nn, c_out), F32)
    else:
        out_specs = pl.BlockSpec((1, c_out, nn), lambda b, k: (b, 0, 0))
        out_shape = jax.ShapeDtypeStruct((B, c_out, nn), F32)
    return pl.pallas_call(
        body,
        grid=(B, nk),
        in_specs=in_specs,
        out_specs=out_specs,
        out_shape=out_shape,
        scratch_shapes=[pltpu.VMEM((c_out, nn), F32)],
        compiler_params=pltpu.CompilerParams(
            dimension_semantics=("parallel", "arbitrary")),
        interpret=_INTERP,
    )(*args)


def _fc1_norm(x, w, bias):
    m = x.shape[0]

    def body(x_ref, w_ref, b_ref, y_ref, xn_ref, xnt_ref):
        y = _dot(x_ref[...], w_ref[...]) + b_ref[...]
        y_ref[...] = y
        n2 = jnp.sum(y * y, axis=1, keepdims=True)
        nrm = jnp.maximum(jnp.sqrt(n2), 1e-12)
        xn = y / nrm
        xn_ref[...] = xn
        xnt_ref[0] = xn.T

    return pl.pallas_call(
        body,
        grid=(B,),
        in_specs=[
            pl.BlockSpec((N_NODES, C), lambda i: (i, 0)),
            pl.BlockSpec((C, C), lambda i: (0, 0)),
            pl.BlockSpec((1, C), lambda i: (0, 0)),
        ],
        out_specs=[
            pl.BlockSpec((N_NODES, C), lambda i: (i, 0)),
            pl.BlockSpec((N_NODES, C), lambda i: (i, 0)),
            pl.BlockSpec((1, C, N_NODES), lambda i: (i, 0, 0)),
        ],
        out_shape=[
            jax.ShapeDtypeStruct((m, C), F32),
            jax.ShapeDtypeStruct((m, C), F32),
            jax.ShapeDtypeStruct((B, C, N_NODES), F32),
        ],
        compiler_params=pltpu.CompilerParams(
            dimension_semantics=("parallel",)),
        interpret=_INTERP,
    )(x, w, bias.reshape(1, C))


_KSCALE = np.float32(2.0 ** 27)
_I32MAX = np.int32(2**31 - 1)


def _topk_idx(xn, xnt):
    m = xn.shape[0]
    nb = N_NODES // TM

    def body(xn_ref, xnt_ref, o_ref):
        t = pl.program_id(0)
        batch = t // nb
        x = xn_ref[...]
        xt = xnt_ref[0]
        sqr = jnp.sum(x * x, axis=1, keepdims=True)
        sqc = jnp.sum(xt * xt, axis=0, keepdims=True)
        ip = _dot(x, xt)
        d = jnp.maximum(sqr - 2.0 * ip + sqc, 0.0)
        ki = (d * _KSCALE).astype(jnp.int32)
        col = lax.broadcasted_iota(jnp.int32, (TM, N_NODES), 1)
        key = jnp.bitwise_or(jnp.bitwise_and(ki, jnp.int32(-4096)), col)
        cols = []
        for _ in range(KNN):
            mv = jnp.min(key, axis=1)
            cols.append(jnp.bitwise_and(mv, jnp.int32(4095)))
            key = jnp.where(key == mv[:, None], _I32MAX, key)
        idx = jnp.stack(cols, axis=1) + batch * N_NODES
        pad = jnp.zeros((TM, 16 - KNN), jnp.int32)
        o_ref[...] = jnp.concatenate([idx, pad], axis=1)

    return pl.pallas_call(
        body,
        grid=(m // TM,),
        in_specs=[
            pl.BlockSpec((TM, C), lambda i: (i, 0)),
            pl.BlockSpec((1, C, N_NODES), lambda i: (i // nb, 0, 0)),
        ],
        out_specs=pl.BlockSpec((TM, 16), lambda i: (i, 0)),
        out_shape=jax.ShapeDtypeStruct((m, 16), jnp.int32),
        compiler_params=pltpu.CompilerParams(
            dimension_semantics=("arbitrary",)),
        interpret=_INTERP,
    )(xn, xnt)


_CHUNK_IDX = 72
_CHUNK_OUT = 8
_N_CHUNKS = (B * N_NODES) // _CHUNK_OUT
_NW = 32
_MAX_T = (_N_CHUNKS + _NW - 1) // _NW


def _sc_gather_max(table, idxf):
    mesh = plsc.VectorSubcoreMesh(core_axis_name="c", subcore_axis_name="s")
    nv = C // 16

    @functools.partial(
        pl.kernel,
        out_type=jax.ShapeDtypeStruct((B * N_NODES, C), F32),
        mesh=mesh,
        scratch_types=[
            pltpu.VMEM((_CHUNK_IDX,), jnp.int32),
            pltpu.VMEM((_CHUNK_IDX,), jnp.int32),
            pltpu.VMEM((_CHUNK_IDX, C), F32),
            pltpu.VMEM((_CHUNK_IDX, C), F32),
            pltpu.VMEM((_CHUNK_OUT, C), F32),
            pltpu.SemaphoreType.DMA,
            pltpu.SemaphoreType.DMA,
        ],
        compiler_params=pltpu.CompilerParams(use_tc_tiling_on_sc=False),
    )
    def k(tab_hbm, idx_hbm, out_hbm, idx0, idx1, rows0, rows1, out_v,
          sem0, sem1):
        wid = lax.axis_index("s") * 2 + lax.axis_index("c")
        idxb = [idx0, idx1]
        rowsb = [rows0, rows1]
        semb = [sem0, sem1]

        pltpu.sync_copy(idx_hbm.at[pl.ds(wid * _CHUNK_IDX, _CHUNK_IDX)], idx0)
        pltpu.make_async_copy(tab_hbm.at[idx0], rows0, sem0).start()

        @pl.loop(0, 2 * ((_MAX_T + 1) // 2), step=2)
        def _(tt):
            for j in range(2):
                t = tt + j
                c = wid + _NW * t

                @pl.when(c < _N_CHUNKS)
                def _():
                    pltpu.make_async_copy(
                        tab_hbm.at[idxb[j]], rowsb[j], semb[j]).wait()
                    cn = wid + _NW * (t + 1)

                    @pl.when(cn < _N_CHUNKS)
                    def _():
                        pltpu.sync_copy(
                            idx_hbm.at[pl.ds(cn * _CHUNK_IDX, _CHUNK_IDX)],
                            idxb[1 - j])
                        pltpu.make_async_copy(
                            tab_hbm.at[idxb[1 - j]], rowsb[1 - j],
                            semb[1 - j]).start()

                    @pl.loop(0, _CHUNK_OUT)
                    def _(nrow):
                        base = nrow * KNN
                        for v in range(nv):
                            sl = pl.ds(v * 16, 16)
                            acc = rowsb[j][base, sl]
                            for r in range(1, KNN):
                                acc = jnp.maximum(acc, rowsb[j][base + r, sl])
                            out_v[nrow, sl] = acc

                    pltpu.sync_copy(
                        out_v, out_hbm.at[pl.ds(c * _CHUNK_OUT, _CHUNK_OUT)])

    return k(table, idxf)


def _block_tail(y, g, x0, wa, wb, bmr, sg, beg, w2, b2, wf1, bf1, wf2, bf2):
    m = y.shape[0]

    def body(y_ref, g_ref, x0_ref, wa_ref, wb_ref, bmr_ref, sg_ref, beg_ref,
             w2_ref, b2_ref, wf1_ref, bf1_ref, wf2_ref, bf2_ref, o_ref):
        yv = y_ref[...]
        diff = g_ref[...] - yv
        z = _dot(yv, wa_ref[...]) + _dot(diff, wb_ref[...]) + bmr_ref[...]
        h = _gelu(z)
        h = _gelu(h * sg_ref[...] + beg_ref[...])
        xm = _dot(h, w2_ref[...]) + b2_ref[...] + x0_ref[...]
        tt = _gelu(_dot(xm, wf1_ref[...]) + bf1_ref[...])
        o_ref[...] = _dot(tt, wf2_ref[...]) + bf2_ref[...] + xm

    vec = lambda a: a.reshape(1, -1)
    row_spec = pl.BlockSpec((TM, C), lambda i: (i, 0))
    w_spec = pl.BlockSpec((C, C), lambda i: (0, 0))
    v_spec = pl.BlockSpec((1, C), lambda i: (0, 0))
    return pl.pallas_call(
        body,
        grid=(m // TM,),
        in_specs=[row_spec, row_spec, row_spec,
                  w_spec, w_spec, v_spec, v_spec, v_spec,
                  w_spec, v_spec, w_spec, v_spec, w_spec, v_spec],
        out_specs=row_spec,
        out_shape=jax.ShapeDtypeStruct((m, C), F32),
        compiler_params=pltpu.CompilerParams(
            dimension_semantics=("parallel",)),
        interpret=_INTERP,
    )(y, g, x0, wa, wb, vec(bmr), vec(sg), vec(beg),
      w2, vec(b2), wf1, vec(bf1), wf2, vec(bf2))


def _head(x, w1, b1, w2, b2):
    def body(x_ref, w1_ref, b1_ref, w2_ref, b2_ref, o_ref):
        xs = x_ref[...]
        mn = jnp.mean(xs.reshape(B, N_NODES, C), axis=1)
        z = _gelu(_dot(mn, w1_ref[...]) + b1_ref[...])
        o_ref[...] = _dot(z, w2_ref[...]) + b2_ref[...]

    n1 = w1.shape[1]
    n2 = w2.shape[1]
    return pl.pallas_call(
        body,
        in_specs=[
            pl.BlockSpec(x.shape, lambda: (0, 0)),
            pl.BlockSpec(w1.shape, lambda: (0, 0)),
            pl.BlockSpec((1, n1), lambda: (0, 0)),
            pl.BlockSpec(w2.shape, lambda: (0, 0)),
            pl.BlockSpec((1, n2), lambda: (0, 0)),
        ],
        out_specs=pl.BlockSpec((B, n2), lambda: (0, 0)),
        out_shape=jax.ShapeDtypeStruct((B, n2), F32),
        interpret=_INTERP,
    )(x, w1, b1.reshape(1, n1), w2, b2.reshape(1, n2))


def _fold(w2d, bias, g, be):
    s = g * _BN_S
    return w2d * s[None, :], bias * s + be


def _s2_weight_cm(w, g):
    o, i = w.shape[0], w.shape[1]
    ws = w * (g * _BN_S)[:, None, None, None]
    wp = jnp.zeros((2, 2, i, 2, 2, o), F32)
    for dy in range(3):
        cy, r = (dy + 1) // 2, (dy + 1) % 2
        for dx in range(3):
            cx, cc = (dx + 1) // 2, (dx + 1) % 2
            wp = wp.at[cy, cx, :, r, cc].set(ws[:, :, dy, dx].T)
    return wp


def kernel(inputs, params):
    p = params
    s = p['stem']

    xs = (inputs.reshape(B, 3, 112, 2, 112, 2)
          .transpose(0, 1, 3, 5, 2, 4).reshape(B, 12, 112, 112))
    xp = jnp.pad(xs, ((0, 0), (0, 0), (1, 0), (1, 0)))
    cat = jnp.concatenate(
        [xp[:, :, cy:cy + 112, cx:cx + 112] for cy in (0, 1)
         for cx in (0, 1)], axis=1)
    p1 = jnp.stack([cat[:, :, r::2, c::2] for r in (0, 1) for c in (0, 1)],
                   axis=2).reshape(B, 48, 4 * N_NODES)
    w1 = _s2_weight_cm(s['W1'], s['g1']).reshape(48, 96).T.reshape(1, 96, 48)
    b1 = s['b1'] * (s['g1'] * _BN_S) + s['be1']
    y1 = _cmm(p1, w1, b1, act=True, nk=1, nn=4 * N_NODES, c_out=96)

    x1 = y1.reshape(B, 384, 56, 56)
    xp2 = jnp.pad(x1, ((0, 0), (0, 0), (1, 0), (1, 0)))
    p2 = jnp.concatenate(
        [xp2[:, :, cy:cy + 56, cx:cx + 56] for cy in (0, 1)
         for cx in (0, 1)], axis=1).reshape(B, 1536, N_NODES)
    w2 = (_s2_weight_cm(s['W2'], s['g2'])
          .reshape(4, 384, 192).transpose(0, 2, 1))
    b2 = s['b2'] * (s['g2'] * _BN_S) + s['be2']
    y2 = _cmm(p2, w2, b2, act=True, nk=4, nn=N_NODES, c_out=C)

    x2 = y2.reshape(B, C, 56, 56)
    xp3 = jnp.pad(x2, ((0, 0), (0, 0), (1, 1), (1, 1)))
    p3 = jnp.concatenate(
        [xp3[:, :, dy:dy + 56, dx:dx + 56] for dy in range(3)
         for dx in range(3)], axis=1).reshape(B, 9 * C, N_NODES)
    s3 = s['g3'] * _BN_S
    w3 = (s['W3'].transpose(2, 3, 1, 0).reshape(9 * C, C) * s3[None, :]
          ).reshape(9, C, C).transpose(0, 2, 1)
    b3 = s['b3'] * s3 + s['be3']
    pos = p['pos_embed'].reshape(C, N_NODES)
    x0 = _cmm(p3, w3, b3, act=False, nk=9, nn=N_NODES, c_out=C,
              pos=pos, transpose_out=True)

    for blk in p['blocks']:
        wf, bf = _fold(blk['fc1_W'][:, :, 0, 0].T, blk['fc1_b'],
                       blk['fc1_g'], blk['fc1_be'])
        y, xn, xnt = _fc1_norm(x0, wf, bf)
        idx16 = _topk_idx(xn, xnt)
        idxf = idx16[:, :KNN].reshape(-1)
        gmax = _sc_gather_max(y, idxf)

        mr = blk['mr_W'][:, :, 0, 0]
        wa = mr[:, 0::2].T
        wb = mr[:, 1::2].T
        sg = blk['gbn_g'] * _BN_S
        beg = blk['gbn_be']
        w2e, b2e = _fold(blk['fc2_W'][:, :, 0, 0].T, blk['fc2_b'],
                         blk['fc2_g'], blk['fc2_be'])
        wf1, bf1 = _fold(blk['ffn1_W'][:, :, 0, 0].T, blk['ffn1_b'],
                         blk['ffn1_g'], blk['ffn1_be'])
        wf2, bf2 = _fold(blk['ffn2_W'][:, :, 0, 0].T, blk['ffn2_b'],
                         blk['ffn2_g'], blk['ffn2_be'])
        x0 = _block_tail(y, gmax, x0, wa, wb, blk['mr_b'], sg, beg,
                         w2e, b2e, wf1, bf1, wf2, bf2)

    h = p['head']
    wh1, bh1 = _fold(h['W1'][:, :, 0, 0].T, h['b1'], h['g1'], h['be1'])
    wh2 = h['W2'][:, :, 0, 0].T
    return _head(x0, wh1, bh1, wh2, h['b2'])


# speedup vs baseline: 1.4165x; 1.4165x over previous
import functools

import jax
import jax.numpy as jnp
import numpy as np
from jax import lax
from jax.experimental import pallas as pl
from jax.experimental.pallas import tpu as pltpu
from jax.experimental.pallas import tpu_sc as plsc

F32 = jnp.float32
_BN_S = np.float32(1.0 / np.sqrt(1.0 + 1e-5))
_INV_SQRT2 = np.float32(1.0 / np.sqrt(2.0))
_PREC = lax.Precision.HIGHEST
_INTERP = False

N_NODES = 3136
B = 2
C = 192
KNN = 9
TM = 784


def _gelu(x):
    return 0.5 * x * (1.0 + lax.erf(x * _INV_SQRT2))


def _dot(a, b, prec=lax.Precision.DEFAULT):
    return jax.lax.dot_general(a, b, (((1,), (0,)), ((), ())),
                               precision=prec, preferred_element_type=F32)


def _cmm(p3, warr, bias, act, nk, nn, c_out, pos=None, transpose_out=False):
    kc = p3.shape[1] // nk

    def body(*refs):
        if pos is not None:
            p_ref, w_ref, b_ref, pos_ref, o_ref, acc_ref = refs
        else:
            p_ref, w_ref, b_ref, o_ref, acc_ref = refs
        k = pl.program_id(1)
        z = _dot(w_ref[0], p_ref[0])

        @pl.when(k == 0)
        def _():
            acc_ref[...] = z

        @pl.when(k > 0)
        def _():
            acc_ref[...] += z

        @pl.when(k == nk - 1)
        def _():
            r = acc_ref[...] + b_ref[...]
            if act:
                r = _gelu(r)
            if pos is not None:
                r = r + pos_ref[...]
            if transpose_out:
                o_ref[...] = r.T
            else:
                o_ref[0] = r

    in_specs = [
        pl.BlockSpec((1, kc, nn), lambda b, k: (b, k, 0)),
        pl.BlockSpec((1, c_out, kc), lambda b, k: (k, 0, 0)),
        pl.BlockSpec((c_out, 1), lambda b, k: (0, 0)),
    ]
    args = [p3, warr, bias.reshape(c_out, 1)]
    if pos is not None:
        in_specs.append(pl.BlockSpec((c_out, nn), lambda b, k: (0, 0)))
        args.append(pos)
    if transpose_out:
        out_specs = pl.BlockSpec((nn, c_out), lambda b, k: (b, 0))
        out_shape = jax.ShapeDtypeStruct((B * nn, c_out), F32)
    else:
        out_specs = pl.BlockSpec((1, c_out, nn), lambda b, k: (b, 0, 0))
        out_shape = jax.ShapeDtypeStruct((B, c_out, nn), F32)
    return pl.pallas_call(
        body,
        grid=(B, nk),
        in_specs=in_specs,
        out_specs=out_specs,
        out_shape=out_shape,
        scratch_shapes=[pltpu.VMEM((c_out, nn), F32)],
        compiler_params=pltpu.CompilerParams(
            dimension_semantics=("parallel", "arbitrary")),
        interpret=_INTERP,
    )(*args)


def _fc1_norm(x, w, bias):
    m = x.shape[0]

    def body(x_ref, w_ref, b_ref, y_ref, xn_ref, xnt_ref):
        y = _dot(x_ref[...], w_ref[...]) + b_ref[...]
        y_ref[...] = y
        n2 = jnp.sum(y * y, axis=1, keepdims=True)
        nrm = jnp.maximum(jnp.sqrt(n2), 1e-12)
        xn = y / nrm
        xn_ref[...] = xn
        xnt_ref[0] = xn.T

    return pl.pallas_call(
        body,
        grid=(B,),
        in_specs=[
            pl.BlockSpec((N_NODES, C), lambda i: (i, 0)),
            pl.BlockSpec((C, C), lambda i: (0, 0)),
            pl.BlockSpec((1, C), lambda i: (0, 0)),
        ],
        out_specs=[
            pl.BlockSpec((N_NODES, C), lambda i: (i, 0)),
            pl.BlockSpec((N_NODES, C), lambda i: (i, 0)),
            pl.BlockSpec((1, C, N_NODES), lambda i: (i, 0, 0)),
        ],
        out_shape=[
            jax.ShapeDtypeStruct((m, C), F32),
            jax.ShapeDtypeStruct((m, C), F32),
            jax.ShapeDtypeStruct((B, C, N_NODES), F32),
        ],
        compiler_params=pltpu.CompilerParams(
            dimension_semantics=("parallel",)),
        interpret=_INTERP,
    )(x, w, bias.reshape(1, C))


_KSCALE = np.float32(2.0 ** 27)
_I32MAX = np.int32(2**31 - 1)


def _topk_idx(xn, xnt):
    m = xn.shape[0]
    nb = N_NODES // TM

    def body(xn_ref, xnt_ref, o_ref):
        t = pl.program_id(0)
        batch = t // nb
        x = xn_ref[...]
        xt = xnt_ref[0]
        sqr = jnp.sum(x * x, axis=1, keepdims=True)
        sqc = jnp.sum(xt * xt, axis=0, keepdims=True)
        ip = _dot(x, xt)
        d = jnp.maximum(sqr - 2.0 * ip + sqc, 0.0)
        ki = (d * _KSCALE).astype(jnp.int32)
        col = lax.broadcasted_iota(jnp.int32, (TM, N_NODES), 1)
        key = jnp.bitwise_or(jnp.bitwise_and(ki, jnp.int32(-4096)), col)
        cols = []
        for _ in range(KNN):
            mv = jnp.min(key, axis=1)
            cols.append(jnp.bitwise_and(mv, jnp.int32(4095)))
            key = jnp.where(key == mv[:, None], _I32MAX, key)
        idx = jnp.stack(cols, axis=1) + batch * N_NODES
        pad = jnp.zeros((TM, 16 - KNN), jnp.int32)
        o_ref[...] = jnp.concatenate([idx, pad], axis=1)

    return pl.pallas_call(
        body,
        grid=(m // TM,),
        in_specs=[
            pl.BlockSpec((TM, C), lambda i: (i, 0)),
            pl.BlockSpec((1, C, N_NODES), lambda i: (i // nb, 0, 0)),
        ],
        out_specs=pl.BlockSpec((TM, 16), lambda i: (i, 0)),
        out_shape=jax.ShapeDtypeStruct((m, 16), jnp.int32),
        compiler_params=pltpu.CompilerParams(
            dimension_semantics=("parallel",)),
        interpret=_INTERP,
    )(xn, xnt)


_CHUNK_IDX = 72
_CHUNK_OUT = 8
_N_CHUNKS = (B * N_NODES) // _CHUNK_OUT
_NW = 32
_MAX_T = (_N_CHUNKS + _NW - 1) // _NW


def _sc_gather_max(table, idxf):
    mesh = plsc.VectorSubcoreMesh(core_axis_name="c", subcore_axis_name="s")
    nv = C // 16

    @functools.partial(
        pl.kernel,
        out_type=jax.ShapeDtypeStruct((B * N_NODES, C), F32),
        mesh=mesh,
        scratch_types=[
            pltpu.VMEM((_CHUNK_IDX,), jnp.int32),
            pltpu.VMEM((_CHUNK_IDX,), jnp.int32),
            pltpu.VMEM((_CHUNK_IDX, C), F32),
            pltpu.VMEM((_CHUNK_IDX, C), F32),
            pltpu.VMEM((_CHUNK_OUT, C), F32),
            pltpu.SemaphoreType.DMA,
            pltpu.SemaphoreType.DMA,
        ],
        compiler_params=pltpu.CompilerParams(use_tc_tiling_on_sc=False),
    )
    def k(tab_hbm, idx_hbm, out_hbm, idx0, idx1, rows0, rows1, out_v,
          sem0, sem1):
        wid = lax.axis_index("s") * 2 + lax.axis_index("c")
        idxb = [idx0, idx1]
        rowsb = [rows0, rows1]
        semb = [sem0, sem1]

        pltpu.sync_copy(idx_hbm.at[pl.ds(wid * _CHUNK_IDX, _CHUNK_IDX)], idx0)
        pltpu.make_async_copy(tab_hbm.at[idx0], rows0, sem0).start()

        @pl.loop(0, 2 * ((_MAX_T + 1) // 2), step=2)
        def _(tt):
            for j in range(2):
                t = tt + j
                c = wid + _NW * t

                @pl.when(c < _N_CHUNKS)
                def _():
                    pltpu.make_async_copy(
                        tab_hbm.at[idxb[j]], rowsb[j], semb[j]).wait()
                    cn = wid + _NW * (t + 1)

                    @pl.when(cn < _N_CHUNKS)
                    def _():
                        pltpu.sync_copy(
                            idx_hbm.at[pl.ds(cn * _CHUNK_IDX, _CHUNK_IDX)],
                            idxb[1 - j])
                        pltpu.make_async_copy(
                            tab_hbm.at[idxb[1 - j]], rowsb[1 - j],
                            semb[1 - j]).start()

                    @pl.loop(0, _CHUNK_OUT)
                    def _(nrow):
                        base = nrow * KNN
                        for v in range(nv):
                            sl = pl.ds(v * 16, 16)
                            acc = rowsb[j][base, sl]
                            for r in range(1, KNN):
                                acc = jnp.maximum(acc, rowsb[j][base + r, sl])
                            out_v[nrow, sl] = acc

                    pltpu.sync_copy(
                        out_v, out_hbm.at[pl.ds(c * _CHUNK_OUT, _CHUNK_OUT)])

    return k(table, idxf)


def _block_tail(y, g, x0, wa, wb, bmr, sg, beg, w2, b2, wf1, bf1, wf2, bf2):
    m = y.shape[0]

    def body(y_ref, g_ref, x0_ref, wa_ref, wb_ref, bmr_ref, sg_ref, beg_ref,
             w2_ref, b2_ref, wf1_ref, bf1_ref, wf2_ref, bf2_ref, o_ref):
        yv = y_ref[...]
        diff = g_ref[...] - yv
        z = _dot(yv, wa_ref[...]) + _dot(diff, wb_ref[...]) + bmr_ref[...]
        h = _gelu(z)
        h = _gelu(h * sg_ref[...] + beg_ref[...])
        xm = _dot(h, w2_ref[...]) + b2_ref[...] + x0_ref[...]
        tt = _gelu(_dot(xm, wf1_ref[...]) + bf1_ref[...])
        o_ref[...] = _dot(tt, wf2_ref[...]) + bf2_ref[...] + xm

    vec = lambda a: a.reshape(1, -1)
    row_spec = pl.BlockSpec((TM, C), lambda i: (i, 0))
    w_spec = pl.BlockSpec((C, C), lambda i: (0, 0))
    v_spec = pl.BlockSpec((1, C), lambda i: (0, 0))
    return pl.pallas_call(
        body,
        grid=(m // TM,),
        in_specs=[row_spec, row_spec, row_spec,
                  w_spec, w_spec, v_spec, v_spec, v_spec,
                  w_spec, v_spec, w_spec, v_spec, w_spec, v_spec],
        out_specs=row_spec,
        out_shape=jax.ShapeDtypeStruct((m, C), F32),
        compiler_params=pltpu.CompilerParams(
            dimension_semantics=("parallel",)),
        interpret=_INTERP,
    )(y, g, x0, wa, wb, vec(bmr), vec(sg), vec(beg),
      w2, vec(b2), wf1, vec(bf1), wf2, vec(bf2))


def _head(x, w1, b1, w2, b2):
    def body(x_ref, w1_ref, b1_ref, w2_ref, b2_ref, o_ref):
        xs = x_ref[...]
        mn = jnp.mean(xs.reshape(B, N_NODES, C), axis=1)
        z = _gelu(_dot(mn, w1_ref[...]) + b1_ref[...])
        o_ref[...] = _dot(z, w2_ref[...]) + b2_ref[...]

    n1 = w1.shape[1]
    n2 = w2.shape[1]
    return pl.pallas_call(
        body,
        in_specs=[
            pl.BlockSpec(x.shape, lambda: (0, 0)),
            pl.BlockSpec(w1.shape, lambda: (0, 0)),
            pl.BlockSpec((1, n1), lambda: (0, 0)),
            pl.BlockSpec(w2.shape, lambda: (0, 0)),
            pl.BlockSpec((1, n2), lambda: (0, 0)),
        ],
        out_specs=pl.BlockSpec((B, n2), lambda: (0, 0)),
        out_shape=jax.ShapeDtypeStruct((B, n2), F32),
        interpret=_INTERP,
    )(x, w1, b1.reshape(1, n1), w2, b2.reshape(1, n2))


def _fold(w2d, bias, g, be):
    s = g * _BN_S
    return w2d * s[None, :], bias * s + be


def _s2_weight_cm(w, g):
    o, i = w.shape[0], w.shape[1]
    ws = w * (g * _BN_S)[:, None, None, None]
    wp = jnp.zeros((2, 2, i, 2, 2, o), F32)
    for dy in range(3):
        cy, r = dy // 2, dy % 2
        for dx in range(3):
            cx, cc = dx // 2, dx % 2
            wp = wp.at[cy, cx, :, r, cc].set(ws[:, :, dy, dx].T)
    return wp


def _s2d_kernel(x):
    _, cch, hp, wp = x.shape
    h2, w2 = hp // 2, wp // 2

    def body(x_ref, o_ref):
        xv = x_ref[0]
        t1 = jnp.transpose(xv, (0, 2, 1))
        for c in (0, 1):
            tc = t1.reshape(cch, w2, 2, hp)[:, :, c, :]
            t2 = jnp.transpose(tc, (0, 2, 1))
            for r in (0, 1):
                o_ref[0, :, r, c] = t2.reshape(cch, h2, 2, w2)[:, :, r, :]

    return pl.pallas_call(
        body,
        grid=(B,),
        in_specs=[pl.BlockSpec((1, cch, hp, wp), lambda b: (b, 0, 0, 0))],
        out_specs=pl.BlockSpec((1, cch, 2, 2, h2, w2),
                               lambda b: (b, 0, 0, 0, 0, 0)),
        out_shape=jax.ShapeDtypeStruct((B, cch, 2, 2, h2, w2), F32),
        compiler_params=pltpu.CompilerParams(
            dimension_semantics=("parallel",)),
        interpret=_INTERP,
    )(x)


def kernel(inputs, params):
    p = params
    s = p['stem']

    xpad = jnp.pad(inputs, ((0, 0), (0, 0), (1, 1), (1, 1)))
    xs = _s2d_kernel(xpad).reshape(B, 12, 113, 113)
    p1 = jnp.concatenate(
        [xs[:, :, cy:cy + 112, cx:cx + 112] for cy in (0, 1)
         for cx in (0, 1)], axis=1).reshape(B, 48, 4 * N_NODES)
    w1 = _s2_weight_cm(s['W1'], s['g1']).reshape(48, 96).T.reshape(1, 96, 48)
    b1 = s['b1'] * (s['g1'] * _BN_S) + s['be1']
    y1 = _cmm(p1, w1, b1, act=True, nk=1, nn=4 * N_NODES, c_out=96)

    x1p = jnp.pad(y1.reshape(B, 96, 112, 112),
                  ((0, 0), (0, 0), (1, 1), (1, 1)))
    x1s = _s2d_kernel(x1p).reshape(B, 384, 57, 57)
    p2 = jnp.concatenate(
        [x1s[:, :, cy:cy + 56, cx:cx + 56] for cy in (0, 1)
         for cx in (0, 1)], axis=1).reshape(B, 1536, N_NODES)
    w2 = (_s2_weight_cm(s['W2'], s['g2'])
          .reshape(4, 384, 192).transpose(0, 2, 1))
    b2 = s['b2'] * (s['g2'] * _BN_S) + s['be2']
    y2 = _cmm(p2, w2, b2, act=True, nk=4, nn=N_NODES, c_out=C)

    x2 = y2.reshape(B, C, 56, 56)
    xp3 = jnp.pad(x2, ((0, 0), (0, 0), (1, 1), (1, 1)))
    p3 = jnp.concatenate(
        [xp3[:, :, dy:dy + 56, dx:dx + 56] for dy in range(3)
         for dx in range(3)], axis=1).reshape(B, 9 * C, N_NODES)
    s3 = s['g3'] * _BN_S
    w3 = (s['W3'].transpose(2, 3, 1, 0).reshape(9 * C, C) * s3[None, :]
          ).reshape(9, C, C).transpose(0, 2, 1)
    b3 = s['b3'] * s3 + s['be3']
    pos = p['pos_embed'].reshape(C, N_NODES)
    x0 = _cmm(p3, w3, b3, act=False, nk=9, nn=N_NODES, c_out=C,
              pos=pos, transpose_out=True)

    for blk in p['blocks']:
        wf, bf = _fold(blk['fc1_W'][:, :, 0, 0].T, blk['fc1_b'],
                       blk['fc1_g'], blk['fc1_be'])
        y, xn, xnt = _fc1_norm(x0, wf, bf)
        idx16 = _topk_idx(xn, xnt)
        idxf = idx16[:, :KNN].reshape(-1)
        gmax = _sc_gather_max(y, idxf)

        mr = blk['mr_W'][:, :, 0, 0]
        wa = mr[:, 0::2].T
        wb = mr[:, 1::2].T
        sg = blk['gbn_g'] * _BN_S
        beg = blk['gbn_be']
        w2e, b2e = _fold(blk['fc2_W'][:, :, 0, 0].T, blk['fc2_b'],
                         blk['fc2_g'], blk['fc2_be'])
        wf1, bf1 = _fold(blk['ffn1_W'][:, :, 0, 0].T, blk['ffn1_b'],
                         blk['ffn1_g'], blk['ffn1_be'])
        wf2, bf2 = _fold(blk['ffn2_W'][:, :, 0, 0].T, blk['ffn2_b'],
                         blk['ffn2_g'], blk['ffn2_be'])
        x0 = _block_tail(y, gmax, x0, wa, wb, blk['mr_b'], sg, beg,
                         w2e, b2e, wf1, bf1, wf2, bf2)

    h = p['head']
    wh1, bh1 = _fold(h['W1'][:, :, 0, 0].T, h['b1'], h['g1'], h['be1'])
    wh2 = h['W2'][:, :, 0, 0].T
    return _head(x0, wh1, bh1, wh2, h['b2'])


# speedup vs baseline: 1.4169x; 1.0003x over previous
import functools

import jax
import jax.numpy as jnp
import numpy as np
from jax import lax
from jax.experimental import pallas as pl
from jax.experimental.pallas import tpu as pltpu
from jax.experimental.pallas import tpu_sc as plsc

F32 = jnp.float32
_BN_S = np.float32(1.0 / np.sqrt(1.0 + 1e-5))
_INV_SQRT2 = np.float32(1.0 / np.sqrt(2.0))
_PREC = lax.Precision.HIGHEST
_INTERP = False

N_NODES = 3136
B = 2
C = 192
KNN = 9
TM = 784


def _gelu(x):
    return 0.5 * x * (1.0 + lax.erf(x * _INV_SQRT2))


def _dot(a, b, prec=lax.Precision.DEFAULT):
    return jax.lax.dot_general(a, b, (((1,), (0,)), ((), ())),
                               precision=prec, preferred_element_type=F32)


def _cmm(p3, warr, bias, act, nk, nn, c_out, pos=None, transpose_out=False):
    kc = p3.shape[1] // nk

    def body(*refs):
        if pos is not None:
            p_ref, w_ref, b_ref, pos_ref, o_ref, acc_ref = refs
        else:
            p_ref, w_ref, b_ref, o_ref, acc_ref = refs
        k = pl.program_id(1)
        z = _dot(w_ref[0], p_ref[0])

        @pl.when(k == 0)
        def _():
            acc_ref[...] = z

        @pl.when(k > 0)
        def _():
            acc_ref[...] += z

        @pl.when(k == nk - 1)
        def _():
            r = acc_ref[...] + b_ref[...]
            if act:
                r = _gelu(r)
            if pos is not None:
                r = r + pos_ref[...]
            if transpose_out:
                o_ref[...] = r.T
            else:
                o_ref[0] = r

    in_specs = [
        pl.BlockSpec((1, kc, nn), lambda b, k: (b, k, 0)),
        pl.BlockSpec((1, c_out, kc), lambda b, k: (k, 0, 0)),
        pl.BlockSpec((c_out, 1), lambda b, k: (0, 0)),
    ]
    args = [p3, warr, bias.reshape(c_out, 1)]
    if pos is not None:
        in_specs.append(pl.BlockSpec((c_out, nn), lambda b, k: (0, 0)))
        args.append(pos)
    if transpose_out:
        out_specs = pl.BlockSpec((nn, c_out), lambda b, k: (b, 0))
        out_shape = jax.ShapeDtypeStruct((B * nn, c_out), F32)
    else:
        out_specs = pl.BlockSpec((1, c_out, nn), lambda b, k: (b, 0, 0))
        out_shape = jax.ShapeDtypeStruct((B, c_out, nn), F32)
    return pl.pallas_call(
        body,
        grid=(B, nk),
        in_specs=in_specs,
        out_specs=out_specs,
        out_shape=out_shape,
        scratch_shapes=[pltpu.VMEM((c_out, nn), F32)],
        compiler_params=pltpu.CompilerParams(
            dimension_semantics=("parallel", "arbitrary")),
        interpret=_INTERP,
    )(*args)


def _fc1_norm(x, w, bias):
    m = x.shape[0]

    def body(x_ref, w_ref, b_ref, y_ref, xn_ref, xnt_ref):
        y = _dot(x_ref[...], w_ref[...]) + b_ref[...]
        y_ref[...] = y
        n2 = jnp.sum(y * y, axis=1, keepdims=True)
        nrm = jnp.maximum(jnp.sqrt(n2), 1e-12)
        xn = y / nrm
        xn_ref[...] = xn
        xnt_ref[0] = xn.T

    return pl.pallas_call(
        body,
        grid=(B,),
        in_specs=[
            pl.BlockSpec((N_NODES, C), lambda i: (i, 0)),
            pl.BlockSpec((C, C), lambda i: (0, 0)),
            pl.BlockSpec((1, C), lambda i: (0, 0)),
        ],
        out_specs=[
            pl.BlockSpec((N_NODES, C), lambda i: (i, 0)),
            pl.BlockSpec((N_NODES, C), lambda i: (i, 0)),
            pl.BlockSpec((1, C, N_NODES), lambda i: (i, 0, 0)),
        ],
        out_shape=[
            jax.ShapeDtypeStruct((m, C), F32),
            jax.ShapeDtypeStruct((m, C), F32),
            jax.ShapeDtypeStruct((B, C, N_NODES), F32),
        ],
        compiler_params=pltpu.CompilerParams(
            dimension_semantics=("parallel",)),
        interpret=_INTERP,
    )(x, w, bias.reshape(1, C))


_KSCALE = np.float32(2.0 ** 27)
_I32MAX = np.int32(2**31 - 1)


def _topk_idx(xn, xnt):
    m = xn.shape[0]
    nb = N_NODES // TM

    def body(xn_ref, xnt_ref, o_ref):
        t = pl.program_id(0)
        batch = t // nb
        x = xn_ref[...]
        xt = xnt_ref[0]
        sqr = jnp.sum(x * x, axis=1, keepdims=True)
        sqc = jnp.sum(xt * xt, axis=0, keepdims=True)
        ip = _dot(x, xt)
        d = jnp.maximum(sqr - 2.0 * ip + sqc, 0.0)
        ki = (d * _KSCALE).astype(jnp.int32)
        col = lax.broadcasted_iota(jnp.int32, (TM, N_NODES), 1)
        key = jnp.bitwise_or(jnp.bitwise_and(ki, jnp.int32(-4096)), col)
        cols = []
        mv = jnp.min(key, axis=1)
        cols.append(jnp.bitwise_and(mv, jnp.int32(4095)))
        for _ in range(KNN - 1):
            mv = jnp.min(jnp.where(key > mv[:, None], key, _I32MAX), axis=1)
            cols.append(jnp.bitwise_and(mv, jnp.int32(4095)))
        idx = jnp.stack(cols, axis=1) + batch * N_NODES
        pad = jnp.zeros((TM, 16 - KNN), jnp.int32)
        o_ref[...] = jnp.concatenate([idx, pad], axis=1)

    return pl.pallas_call(
        body,
        grid=(m // TM,),
        in_specs=[
            pl.BlockSpec((TM, C), lambda i: (i, 0)),
            pl.BlockSpec((1, C, N_NODES), lambda i: (i // nb, 0, 0)),
        ],
        out_specs=pl.BlockSpec((TM, 16), lambda i: (i, 0)),
        out_shape=jax.ShapeDtypeStruct((m, 16), jnp.int32),
        compiler_params=pltpu.CompilerParams(
            dimension_semantics=("parallel",)),
        interpret=_INTERP,
    )(xn, xnt)


_CHUNK_IDX = 72
_CHUNK_OUT = 8
_N_CHUNKS = (B * N_NODES) // _CHUNK_OUT
_NW = 32
_MAX_T = (_N_CHUNKS + _NW - 1) // _NW


def _sc_gather_max(table, idxf):
    mesh = plsc.VectorSubcoreMesh(core_axis_name="c", subcore_axis_name="s")
    nv = C // 16

    @functools.partial(
        pl.kernel,
        out_type=jax.ShapeDtypeStruct((B * N_NODES, C), F32),
        mesh=mesh,
        scratch_types=[
            pltpu.VMEM((_CHUNK_IDX,), jnp.int32),
            pltpu.VMEM((_CHUNK_IDX,), jnp.int32),
            pltpu.VMEM((_CHUNK_IDX, C), F32),
            pltpu.VMEM((_CHUNK_IDX, C), F32),
            pltpu.VMEM((_CHUNK_OUT, C), F32),
            pltpu.SemaphoreType.DMA,
            pltpu.SemaphoreType.DMA,
        ],
        compiler_params=pltpu.CompilerParams(use_tc_tiling_on_sc=False),
    )
    def k(tab_hbm, idx_hbm, out_hbm, idx0, idx1, rows0, rows1, out_v,
          sem0, sem1):
        wid = lax.axis_index("s") * 2 + lax.axis_index("c")
        idxb = [idx0, idx1]
        rowsb = [rows0, rows1]
        semb = [sem0, sem1]

        pltpu.sync_copy(idx_hbm.at[pl.ds(wid * _CHUNK_IDX, _CHUNK_IDX)], idx0)
        pltpu.make_async_copy(tab_hbm.at[idx0], rows0, sem0).start()

        @pl.loop(0, 2 * ((_MAX_T + 1) // 2), step=2)
        def _(tt):
            for j in range(2):
                t = tt + j
                c = wid + _NW * t

                @pl.when(c < _N_CHUNKS)
                def _():
                    pltpu.make_async_copy(
                        tab_hbm.at[idxb[j]], rowsb[j], semb[j]).wait()
                    cn = wid + _NW * (t + 1)

                    @pl.when(cn < _N_CHUNKS)
                    def _():
                        pltpu.sync_copy(
                            idx_hbm.at[pl.ds(cn * _CHUNK_IDX, _CHUNK_IDX)],
                            idxb[1 - j])
                        pltpu.make_async_copy(
                            tab_hbm.at[idxb[1 - j]], rowsb[1 - j],
                            semb[1 - j]).start()

                    @pl.loop(0, _CHUNK_OUT)
                    def _(nrow):
                        base = nrow * KNN
                        for v in range(nv):
                            sl = pl.ds(v * 16, 16)
                            acc = rowsb[j][base, sl]
                            for r in range(1, KNN):
                                acc = jnp.maximum(acc, rowsb[j][base + r, sl])
                            out_v[nrow, sl] = acc

                    pltpu.sync_copy(
                        out_v, out_hbm.at[pl.ds(c * _CHUNK_OUT, _CHUNK_OUT)])

    return k(table, idxf)


def _block_tail(y, g, x0, wa, wb, bmr, sg, beg, w2, b2, wf1, bf1, wf2, bf2):
    m = y.shape[0]

    def body(y_ref, g_ref, x0_ref, wa_ref, wb_ref, bmr_ref, sg_ref, beg_ref,
             w2_ref, b2_ref, wf1_ref, bf1_ref, wf2_ref, bf2_ref, o_ref):
        yv = y_ref[...]
        diff = g_ref[...] - yv
        z = _dot(yv, wa_ref[...]) + _dot(diff, wb_ref[...]) + bmr_ref[...]
        h = _gelu(z)
        h = _gelu(h * sg_ref[...] + beg_ref[...])
        xm = _dot(h, w2_ref[...]) + b2_ref[...] + x0_ref[...]
        tt = _gelu(_dot(xm, wf1_ref[...]) + bf1_ref[...])
        o_ref[...] = _dot(tt, wf2_ref[...]) + bf2_ref[...] + xm

    vec = lambda a: a.reshape(1, -1)
    row_spec = pl.BlockSpec((TM, C), lambda i: (i, 0))
    w_spec = pl.BlockSpec((C, C), lambda i: (0, 0))
    v_spec = pl.BlockSpec((1, C), lambda i: (0, 0))
    return pl.pallas_call(
        body,
        grid=(m // TM,),
        in_specs=[row_spec, row_spec, row_spec,
                  w_spec, w_spec, v_spec, v_spec, v_spec,
                  w_spec, v_spec, w_spec, v_spec, w_spec, v_spec],
        out_specs=row_spec,
        out_shape=jax.ShapeDtypeStruct((m, C), F32),
        compiler_params=pltpu.CompilerParams(
            dimension_semantics=("parallel",)),
        interpret=_INTERP,
    )(y, g, x0, wa, wb, vec(bmr), vec(sg), vec(beg),
      w2, vec(b2), wf1, vec(bf1), wf2, vec(bf2))


def _head(x, w1, b1, w2, b2):
    def body(x_ref, w1_ref, b1_ref, w2_ref, b2_ref, o_ref):
        xs = x_ref[...]
        mn = jnp.mean(xs.reshape(B, N_NODES, C), axis=1)
        z = _gelu(_dot(mn, w1_ref[...]) + b1_ref[...])
        o_ref[...] = _dot(z, w2_ref[...]) + b2_ref[...]

    n1 = w1.shape[1]
    n2 = w2.shape[1]
    return pl.pallas_call(
        body,
        in_specs=[
            pl.BlockSpec(x.shape, lambda: (0, 0)),
            pl.BlockSpec(w1.shape, lambda: (0, 0)),
            pl.BlockSpec((1, n1), lambda: (0, 0)),
            pl.BlockSpec(w2.shape, lambda: (0, 0)),
            pl.BlockSpec((1, n2), lambda: (0, 0)),
        ],
        out_specs=pl.BlockSpec((B, n2), lambda: (0, 0)),
        out_shape=jax.ShapeDtypeStruct((B, n2), F32),
        interpret=_INTERP,
    )(x, w1, b1.reshape(1, n1), w2, b2.reshape(1, n2))


def _fold(w2d, bias, g, be):
    s = g * _BN_S
    return w2d * s[None, :], bias * s + be


def _s2_weight_cm(w, g):
    o, i = w.shape[0], w.shape[1]
    ws = w * (g * _BN_S)[:, None, None, None]
    wp = jnp.zeros((2, 2, i, 2, 2, o), F32)
    for dy in range(3):
        cy, r = dy // 2, dy % 2
        for dx in range(3):
            cx, cc = dx // 2, dx % 2
            wp = wp.at[cy, cx, :, r, cc].set(ws[:, :, dy, dx].T)
    return wp


def _s2d_kernel(x):
    _, cch, hp, wp = x.shape
    h2, w2 = hp // 2, wp // 2

    def body(x_ref, o_ref):
        xv = x_ref[0]
        t1 = jnp.transpose(xv, (0, 2, 1))
        for c in (0, 1):
            tc = t1.reshape(cch, w2, 2, hp)[:, :, c, :]
            t2 = jnp.transpose(tc, (0, 2, 1))
            for r in (0, 1):
                o_ref[0, :, r, c] = t2.reshape(cch, h2, 2, w2)[:, :, r, :]

    return pl.pallas_call(
        body,
        grid=(B,),
        in_specs=[pl.BlockSpec((1, cch, hp, wp), lambda b: (b, 0, 0, 0))],
        out_specs=pl.BlockSpec((1, cch, 2, 2, h2, w2),
                               lambda b: (b, 0, 0, 0, 0, 0)),
        out_shape=jax.ShapeDtypeStruct((B, cch, 2, 2, h2, w2), F32),
        compiler_params=pltpu.CompilerParams(
            dimension_semantics=("parallel",)),
        interpret=_INTERP,
    )(x)


def kernel(inputs, params):
    p = params
    s = p['stem']

    xpad = jnp.pad(inputs, ((0, 0), (0, 0), (1, 1), (1, 1)))
    xs = _s2d_kernel(xpad).reshape(B, 12, 113, 113)
    p1 = jnp.concatenate(
        [xs[:, :, cy:cy + 112, cx:cx + 112] for cy in (0, 1)
         for cx in (0, 1)], axis=1).reshape(B, 48, 4 * N_NODES)
    w1 = _s2_weight_cm(s['W1'], s['g1']).reshape(48, 96).T.reshape(1, 96, 48)
    b1 = s['b1'] * (s['g1'] * _BN_S) + s['be1']
    y1 = _cmm(p1, w1, b1, act=True, nk=1, nn=4 * N_NODES, c_out=96)

    x1p = jnp.pad(y1.reshape(B, 96, 112, 112),
                  ((0, 0), (0, 0), (1, 1), (1, 1)))
    x1s = _s2d_kernel(x1p).reshape(B, 384, 57, 57)
    p2 = jnp.concatenate(
        [x1s[:, :, cy:cy + 56, cx:cx + 56] for cy in (0, 1)
         for cx in (0, 1)], axis=1).reshape(B, 1536, N_NODES)
    w2 = (_s2_weight_cm(s['W2'], s['g2'])
          .reshape(4, 384, 192).transpose(0, 2, 1))
    b2 = s['b2'] * (s['g2'] * _BN_S) + s['be2']
    y2 = _cmm(p2, w2, b2, act=True, nk=4, nn=N_NODES, c_out=C)

    x2 = y2.reshape(B, C, 56, 56)
    xp3 = jnp.pad(x2, ((0, 0), (0, 0), (1, 1), (1, 1)))
    p3 = jnp.concatenate(
        [xp3[:, :, dy:dy + 56, dx:dx + 56] for dy in range(3)
         for dx in range(3)], axis=1).reshape(B, 9 * C, N_NODES)
    s3 = s['g3'] * _BN_S
    w3 = (s['W3'].transpose(2, 3, 1, 0).reshape(9 * C, C) * s3[None, :]
          ).reshape(9, C, C).transpose(0, 2, 1)
    b3 = s['b3'] * s3 + s['be3']
    pos = p['pos_embed'].reshape(C, N_NODES)
    x0 = _cmm(p3, w3, b3, act=False, nk=9, nn=N_NODES, c_out=C,
              pos=pos, transpose_out=True)

    for blk in p['blocks']:
        wf, bf = _fold(blk['fc1_W'][:, :, 0, 0].T, blk['fc1_b'],
                       blk['fc1_g'], blk['fc1_be'])
        y, xn, xnt = _fc1_norm(x0, wf, bf)
        idx16 = _topk_idx(xn, xnt)
        idxf = idx16[:, :KNN].reshape(-1)
        gmax = _sc_gather_max(y, idxf)

        mr = blk['mr_W'][:, :, 0, 0]
        wa = mr[:, 0::2].T
        wb = mr[:, 1::2].T
        sg = blk['gbn_g'] * _BN_S
        beg = blk['gbn_be']
        w2e, b2e = _fold(blk['fc2_W'][:, :, 0, 0].T, blk['fc2_b'],
                         blk['fc2_g'], blk['fc2_be'])
        wf1, bf1 = _fold(blk['ffn1_W'][:, :, 0, 0].T, blk['ffn1_b'],
                         blk['ffn1_g'], blk['ffn1_be'])
        wf2, bf2 = _fold(blk['ffn2_W'][:, :, 0, 0].T, blk['ffn2_b'],
                         blk['ffn2_g'], blk['ffn2_be'])
        x0 = _block_tail(y, gmax, x0, wa, wb, blk['mr_b'], sg, beg,
                         w2e, b2e, wf1, bf1, wf2, bf2)

    h = p['head']
    wh1, bh1 = _fold(h['W1'][:, :, 0, 0].T, h['b1'], h['g1'], h['be1'])
    wh2 = h['W2'][:, :, 0, 0].T
    return _head(x0, wh1, bh1, wh2, h['b2'])


# speedup vs baseline: 1.4586x; 1.0294x over previous
import functools

import jax
import jax.numpy as jnp
import numpy as np
from jax import lax
from jax.experimental import pallas as pl
from jax.experimental.pallas import tpu as pltpu
from jax.experimental.pallas import tpu_sc as plsc

F32 = jnp.float32
_BN_S = np.float32(1.0 / np.sqrt(1.0 + 1e-5))
_INV_SQRT2 = np.float32(1.0 / np.sqrt(2.0))
_PREC = lax.Precision.HIGHEST
_INTERP = False

N_NODES = 3136
B = 2
C = 192
KNN = 9
TM = 784


def _gelu(x):
    return 0.5 * x * (1.0 + lax.erf(x * _INV_SQRT2))


def _dot(a, b, prec=lax.Precision.DEFAULT):
    return jax.lax.dot_general(a, b, (((1,), (0,)), ((), ())),
                               precision=prec, preferred_element_type=F32)


def _cmm(p3, warr, bias, act, nk, nn, c_out, pos=None, transpose_out=False):
    kc = p3.shape[1] // nk

    def body(*refs):
        if pos is not None:
            p_ref, w_ref, b_ref, pos_ref, o_ref, acc_ref = refs
        else:
            p_ref, w_ref, b_ref, o_ref, acc_ref = refs
        k = pl.program_id(1)
        z = _dot(w_ref[0], p_ref[0])

        @pl.when(k == 0)
        def _():
            acc_ref[...] = z

        @pl.when(k > 0)
        def _():
            acc_ref[...] += z

        @pl.when(k == nk - 1)
        def _():
            r = acc_ref[...] + b_ref[...]
            if act:
                r = _gelu(r)
            if pos is not None:
                r = r + pos_ref[...]
            if transpose_out:
                o_ref[...] = r.T
            else:
                o_ref[0] = r

    in_specs = [
        pl.BlockSpec((1, kc, nn), lambda b, k: (b, k, 0)),
        pl.BlockSpec((1, c_out, kc), lambda b, k: (k, 0, 0)),
        pl.BlockSpec((c_out, 1), lambda b, k: (0, 0)),
    ]
    args = [p3, warr, bias.reshape(c_out, 1)]
    if pos is not None:
        in_specs.append(pl.BlockSpec((c_out, nn), lambda b, k: (0, 0)))
        args.append(pos)
    if transpose_out:
        out_specs = pl.BlockSpec((nn, c_out), lambda b, k: (b, 0))
        out_shape = jax.ShapeDtypeStruct((B * nn, c_out), F32)
    else:
        out_specs = pl.BlockSpec((1, c_out, nn), lambda b, k: (b, 0, 0))
        out_shape = jax.ShapeDtypeStruct((B, c_out, nn), F32)
    return pl.pallas_call(
        body,
        grid=(B, nk),
        in_specs=in_specs,
        out_specs=out_specs,
        out_shape=out_shape,
        scratch_shapes=[pltpu.VMEM((c_out, nn), F32)],
        compiler_params=pltpu.CompilerParams(
            dimension_semantics=("parallel", "arbitrary")),
        interpret=_INTERP,
    )(*args)


def _fc1_norm(x, w, bias):
    m = x.shape[0]

    def body(x_ref, w_ref, b_ref, y_ref, xn_ref, xnt_ref):
        y = _dot(x_ref[...], w_ref[...]) + b_ref[...]
        y_ref[...] = y
        n2 = jnp.sum(y * y, axis=1, keepdims=True)
        nrm = jnp.maximum(jnp.sqrt(n2), 1e-12)
        xn = y / nrm
        xn_ref[...] = xn
        xnt_ref[0] = xn.T

    return pl.pallas_call(
        body,
        grid=(B,),
        in_specs=[
            pl.BlockSpec((N_NODES, C), lambda i: (i, 0)),
            pl.BlockSpec((C, C), lambda i: (0, 0)),
            pl.BlockSpec((1, C), lambda i: (0, 0)),
        ],
        out_specs=[
            pl.BlockSpec((N_NODES, C), lambda i: (i, 0)),
            pl.BlockSpec((N_NODES, C), lambda i: (i, 0)),
            pl.BlockSpec((1, C, N_NODES), lambda i: (i, 0, 0)),
        ],
        out_shape=[
            jax.ShapeDtypeStruct((m, C), F32),
            jax.ShapeDtypeStruct((m, C), F32),
            jax.ShapeDtypeStruct((B, C, N_NODES), F32),
        ],
        compiler_params=pltpu.CompilerParams(
            dimension_semantics=("parallel",)),
        interpret=_INTERP,
    )(x, w, bias.reshape(1, C))


_KSCALE = np.float32(2.0 ** 27)
_I32MAX = np.int32(2**31 - 1)


def _topk_idx(xn, xnt, batch):
    nb = N_NODES // TM

    def body(xn_ref, xnt_ref, o_ref):
        x = xn_ref[...]
        xt = xnt_ref[0]
        sqr = jnp.sum(x * x, axis=1, keepdims=True)
        sqc = jnp.sum(xt * xt, axis=0, keepdims=True)
        ip = _dot(x, xt)
        d = jnp.maximum(sqr - 2.0 * ip + sqc, 0.0)
        ki = (d * _KSCALE).astype(jnp.int32)
        col = lax.broadcasted_iota(jnp.int32, (TM, N_NODES), 1)
        key = jnp.bitwise_or(jnp.bitwise_and(ki, jnp.int32(-4096)), col)
        cols = []
        mv = jnp.min(key, axis=1)
        cols.append(jnp.bitwise_and(mv, jnp.int32(4095)))
        for _ in range(KNN - 1):
            mv = jnp.min(jnp.where(key > mv[:, None], key, _I32MAX), axis=1)
            cols.append(jnp.bitwise_and(mv, jnp.int32(4095)))
        idx = jnp.stack(cols, axis=1) + batch * N_NODES
        pad = jnp.zeros((TM, 16 - KNN), jnp.int32)
        o_ref[...] = jnp.concatenate([idx, pad], axis=1)

    return pl.pallas_call(
        body,
        grid=(nb,),
        in_specs=[
            pl.BlockSpec((TM, C), lambda i: (batch * nb + i, 0)),
            pl.BlockSpec((1, C, N_NODES), lambda i: (batch, 0, 0)),
        ],
        out_specs=pl.BlockSpec((TM, 16), lambda i: (i, 0)),
        out_shape=jax.ShapeDtypeStruct((N_NODES, 16), jnp.int32),
        compiler_params=pltpu.CompilerParams(
            dimension_semantics=("parallel",)),
        interpret=_INTERP,
    )(xn, xnt)


_CHUNK_IDX = 72
_CHUNK_OUT = 8
_NW = 32


def _sc_gather_max(table, idxf, n_nodes):
    mesh = plsc.VectorSubcoreMesh(core_axis_name="c", subcore_axis_name="s")
    nv = C // 16
    _N_CHUNKS = n_nodes // _CHUNK_OUT
    _MAX_T = (_N_CHUNKS + _NW - 1) // _NW

    @functools.partial(
        pl.kernel,
        out_type=jax.ShapeDtypeStruct((n_nodes, C), F32),
        mesh=mesh,
        scratch_types=[
            pltpu.VMEM((_CHUNK_IDX,), jnp.int32),
            pltpu.VMEM((_CHUNK_IDX,), jnp.int32),
            pltpu.VMEM((_CHUNK_IDX, C), F32),
            pltpu.VMEM((_CHUNK_IDX, C), F32),
            pltpu.VMEM((_CHUNK_OUT, C), F32),
            pltpu.SemaphoreType.DMA,
            pltpu.SemaphoreType.DMA,
        ],
        compiler_params=pltpu.CompilerParams(use_tc_tiling_on_sc=False),
    )
    def k(tab_hbm, idx_hbm, out_hbm, idx0, idx1, rows0, rows1, out_v,
          sem0, sem1):
        wid = lax.axis_index("s") * 2 + lax.axis_index("c")
        idxb = [idx0, idx1]
        rowsb = [rows0, rows1]
        semb = [sem0, sem1]

        pltpu.sync_copy(idx_hbm.at[pl.ds(wid * _CHUNK_IDX, _CHUNK_IDX)], idx0)
        pltpu.make_async_copy(tab_hbm.at[idx0], rows0, sem0).start()

        @pl.loop(0, 2 * ((_MAX_T + 1) // 2), step=2)
        def _(tt):
            for j in range(2):
                t = tt + j
                c = wid + _NW * t

                @pl.when(c < _N_CHUNKS)
                def _():
                    pltpu.make_async_copy(
                        tab_hbm.at[idxb[j]], rowsb[j], semb[j]).wait()
                    cn = wid + _NW * (t + 1)

                    @pl.when(cn < _N_CHUNKS)
                    def _():
                        pltpu.sync_copy(
                            idx_hbm.at[pl.ds(cn * _CHUNK_IDX, _CHUNK_IDX)],
                            idxb[1 - j])
                        pltpu.make_async_copy(
                            tab_hbm.at[idxb[1 - j]], rowsb[1 - j],
                            semb[1 - j]).start()

                    @pl.loop(0, _CHUNK_OUT)
                    def _(nrow):
                        base = nrow * KNN
                        for v in range(nv):
                            sl = pl.ds(v * 16, 16)
                            acc = rowsb[j][base, sl]
                            for r in range(1, KNN):
                                acc = jnp.maximum(acc, rowsb[j][base + r, sl])
                            out_v[nrow, sl] = acc

                    pltpu.sync_copy(
                        out_v, out_hbm.at[pl.ds(c * _CHUNK_OUT, _CHUNK_OUT)])

    return k(table, idxf)


def _block_tail(y, g, x0, wa, wb, bmr, sg, beg, w2, b2, wf1, bf1, wf2, bf2):
    m = y.shape[0]

    def body(y_ref, g_ref, x0_ref, wa_ref, wb_ref, bmr_ref, sg_ref, beg_ref,
             w2_ref, b2_ref, wf1_ref, bf1_ref, wf2_ref, bf2_ref, o_ref):
        yv = y_ref[...]
        diff = g_ref[...] - yv
        z = _dot(yv, wa_ref[...]) + _dot(diff, wb_ref[...]) + bmr_ref[...]
        h = _gelu(z)
        h = _gelu(h * sg_ref[...] + beg_ref[...])
        xm = _dot(h, w2_ref[...]) + b2_ref[...] + x0_ref[...]
        tt = _gelu(_dot(xm, wf1_ref[...]) + bf1_ref[...])
        o_ref[...] = _dot(tt, wf2_ref[...]) + bf2_ref[...] + xm

    vec = lambda a: a.reshape(1, -1)
    row_spec = pl.BlockSpec((TM, C), lambda i: (i, 0))
    w_spec = pl.BlockSpec((C, C), lambda i: (0, 0))
    v_spec = pl.BlockSpec((1, C), lambda i: (0, 0))
    return pl.pallas_call(
        body,
        grid=(m // TM,),
        in_specs=[row_spec, row_spec, row_spec,
                  w_spec, w_spec, v_spec, v_spec, v_spec,
                  w_spec, v_spec, w_spec, v_spec, w_spec, v_spec],
        out_specs=row_spec,
        out_shape=jax.ShapeDtypeStruct((m, C), F32),
        compiler_params=pltpu.CompilerParams(
            dimension_semantics=("parallel",)),
        interpret=_INTERP,
    )(y, g, x0, wa, wb, vec(bmr), vec(sg), vec(beg),
      w2, vec(b2), wf1, vec(bf1), wf2, vec(bf2))


def _head(x, w1, b1, w2, b2):
    def body(x_ref, w1_ref, b1_ref, w2_ref, b2_ref, o_ref):
        xs = x_ref[...]
        mn = jnp.mean(xs.reshape(B, N_NODES, C), axis=1)
        z = _gelu(_dot(mn, w1_ref[...]) + b1_ref[...])
        o_ref[...] = _dot(z, w2_ref[...]) + b2_ref[...]

    n1 = w1.shape[1]
    n2 = w2.shape[1]
    return pl.pallas_call(
        body,
        in_specs=[
            pl.BlockSpec(x.shape, lambda: (0, 0)),
            pl.BlockSpec(w1.shape, lambda: (0, 0)),
            pl.BlockSpec((1, n1), lambda: (0, 0)),
            pl.BlockSpec(w2.shape, lambda: (0, 0)),
            pl.BlockSpec((1, n2), lambda: (0, 0)),
        ],
        out_specs=pl.BlockSpec((B, n2), lambda: (0, 0)),
        out_shape=jax.ShapeDtypeStruct((B, n2), F32),
        interpret=_INTERP,
    )(x, w1, b1.reshape(1, n1), w2, b2.reshape(1, n2))


def _fold(w2d, bias, g, be):
    s = g * _BN_S
    return w2d * s[None, :], bias * s + be


def _s2_weight_cm(w, g):
    o, i = w.shape[0], w.shape[1]
    ws = w * (g * _BN_S)[:, None, None, None]
    wp = jnp.zeros((2, 2, i, 2, 2, o), F32)
    for dy in range(3):
        cy, r = dy // 2, dy % 2
        for dx in range(3):
            cx, cc = dx // 2, dx % 2
            wp = wp.at[cy, cx, :, r, cc].set(ws[:, :, dy, dx].T)
    return wp


def _s2d_kernel(x):
    _, cch, hp, wp = x.shape
    h2, w2 = hp // 2, wp // 2

    def body(x_ref, o_ref):
        xv = x_ref[0]
        t1 = jnp.transpose(xv, (0, 2, 1))
        for c in (0, 1):
            tc = t1.reshape(cch, w2, 2, hp)[:, :, c, :]
            t2 = jnp.transpose(tc, (0, 2, 1))
            for r in (0, 1):
                o_ref[0, :, r, c] = t2.reshape(cch, h2, 2, w2)[:, :, r, :]

    return pl.pallas_call(
        body,
        grid=(B,),
        in_specs=[pl.BlockSpec((1, cch, hp, wp), lambda b: (b, 0, 0, 0))],
        out_specs=pl.BlockSpec((1, cch, 2, 2, h2, w2),
                               lambda b: (b, 0, 0, 0, 0, 0)),
        out_shape=jax.ShapeDtypeStruct((B, cch, 2, 2, h2, w2), F32),
        compiler_params=pltpu.CompilerParams(
            dimension_semantics=("parallel",)),
        interpret=_INTERP,
    )(x)


def kernel(inputs, params):
    p = params
    s = p['stem']

    xpad = jnp.pad(inputs, ((0, 0), (0, 0), (1, 1), (1, 1)))
    xs = _s2d_kernel(xpad).reshape(B, 12, 113, 113)
    p1 = jnp.concatenate(
        [xs[:, :, cy:cy + 112, cx:cx + 112] for cy in (0, 1)
         for cx in (0, 1)], axis=1).reshape(B, 48, 4 * N_NODES)
    w1 = _s2_weight_cm(s['W1'], s['g1']).reshape(48, 96).T.reshape(1, 96, 48)
    b1 = s['b1'] * (s['g1'] * _BN_S) + s['be1']
    y1 = _cmm(p1, w1, b1, act=True, nk=1, nn=4 * N_NODES, c_out=96)

    x1p = jnp.pad(y1.reshape(B, 96, 112, 112),
                  ((0, 0), (0, 0), (1, 1), (1, 1)))
    x1s = _s2d_kernel(x1p).reshape(B, 384, 57, 57)
    p2 = jnp.concatenate(
        [x1s[:, :, cy:cy + 56, cx:cx + 56] for cy in (0, 1)
         for cx in (0, 1)], axis=1).reshape(B, 1536, N_NODES)
    w2 = (_s2_weight_cm(s['W2'], s['g2'])
          .reshape(4, 384, 192).transpose(0, 2, 1))
    b2 = s['b2'] * (s['g2'] * _BN_S) + s['be2']
    y2 = _cmm(p2, w2, b2, act=True, nk=4, nn=N_NODES, c_out=C)

    x2 = y2.reshape(B, C, 56, 56)
    xp3 = jnp.pad(x2, ((0, 0), (0, 0), (1, 1), (1, 1)))
    p3 = jnp.concatenate(
        [xp3[:, :, dy:dy + 56, dx:dx + 56] for dy in range(3)
         for dx in range(3)], axis=1).reshape(B, 9 * C, N_NODES)
    s3 = s['g3'] * _BN_S
    w3 = (s['W3'].transpose(2, 3, 1, 0).reshape(9 * C, C) * s3[None, :]
          ).reshape(9, C, C).transpose(0, 2, 1)
    b3 = s['b3'] * s3 + s['be3']
    pos = p['pos_embed'].reshape(C, N_NODES)
    x0 = _cmm(p3, w3, b3, act=False, nk=9, nn=N_NODES, c_out=C,
              pos=pos, transpose_out=True)

    for blk in p['blocks']:
        wf, bf = _fold(blk['fc1_W'][:, :, 0, 0].T, blk['fc1_b'],
                       blk['fc1_g'], blk['fc1_be'])
        y, xn, xnt = _fc1_norm(x0, wf, bf)
        gs = []
        for b in range(B):
            idx16 = _topk_idx(xn, xnt, b)
            idxf = idx16[:, :KNN].reshape(-1)
            gs.append(_sc_gather_max(y, idxf, N_NODES))
        gmax = jnp.concatenate(gs, axis=0)

        mr = blk['mr_W'][:, :, 0, 0]
        wa = mr[:, 0::2].T
        wb = mr[:, 1::2].T
        sg = blk['gbn_g'] * _BN_S
        beg = blk['gbn_be']
        w2e, b2e = _fold(blk['fc2_W'][:, :, 0, 0].T, blk['fc2_b'],
                         blk['fc2_g'], blk['fc2_be'])
        wf1, bf1 = _fold(blk['ffn1_W'][:, :, 0, 0].T, blk['ffn1_b'],
                         blk['ffn1_g'], blk['ffn1_be'])
        wf2, bf2 = _fold(blk['ffn2_W'][:, :, 0, 0].T, blk['ffn2_b'],
                         blk['ffn2_g'], blk['ffn2_be'])
        x0 = _block_tail(y, gmax, x0, wa, wb, blk['mr_b'], sg, beg,
                         w2e, b2e, wf1, bf1, wf2, bf2)

    h = p['head']
    wh1, bh1 = _fold(h['W1'][:, :, 0, 0].T, h['b1'], h['g1'], h['be1'])
    wh2 = h['W2'][:, :, 0, 0].T
    return _head(x0, wh1, bh1, wh2, h['b2'])


# speedup vs baseline: 1.6950x; 1.1621x over previous
import functools

import jax
import jax.numpy as jnp
import numpy as np
from jax import lax
from jax.experimental import pallas as pl
from jax.experimental.pallas import tpu as pltpu
from jax.experimental.pallas import tpu_sc as plsc

F32 = jnp.float32
_BN_S = np.float32(1.0 / np.sqrt(1.0 + 1e-5))
_INV_SQRT2 = np.float32(1.0 / np.sqrt(2.0))
_PREC = lax.Precision.HIGHEST
_INTERP = False

N_NODES = 3136
B = 2
C = 192
KNN = 9
TM = 784


def _gelu(x):
    return 0.5 * x * (1.0 + lax.erf(x * _INV_SQRT2))


def _dot(a, b, prec=lax.Precision.DEFAULT):
    return jax.lax.dot_general(a, b, (((1,), (0,)), ((), ())),
                               precision=prec, preferred_element_type=F32)


def _cmm(p3, warr, bias, act, nk, nn, c_out, pos=None, transpose_out=False):
    kc = p3.shape[1] // nk

    def body(*refs):
        if pos is not None:
            p_ref, w_ref, b_ref, pos_ref, o_ref, acc_ref = refs
        else:
            p_ref, w_ref, b_ref, o_ref, acc_ref = refs
        k = pl.program_id(1)
        z = _dot(w_ref[0], p_ref[0])

        @pl.when(k == 0)
        def _():
            acc_ref[...] = z

        @pl.when(k > 0)
        def _():
            acc_ref[...] += z

        @pl.when(k == nk - 1)
        def _():
            r = acc_ref[...] + b_ref[...]
            if act:
                r = _gelu(r)
            if pos is not None:
                r = r + pos_ref[...]
            if transpose_out:
                o_ref[...] = r.T
            else:
                o_ref[0] = r

    in_specs = [
        pl.BlockSpec((1, kc, nn), lambda b, k: (b, k, 0)),
        pl.BlockSpec((1, c_out, kc), lambda b, k: (k, 0, 0)),
        pl.BlockSpec((c_out, 1), lambda b, k: (0, 0)),
    ]
    args = [p3, warr, bias.reshape(c_out, 1)]
    if pos is not None:
        in_specs.append(pl.BlockSpec((c_out, nn), lambda b, k: (0, 0)))
        args.append(pos)
    if transpose_out:
        out_specs = pl.BlockSpec((nn, c_out), lambda b, k: (b, 0))
        out_shape = jax.ShapeDtypeStruct((B * nn, c_out), F32)
    else:
        out_specs = pl.BlockSpec((1, c_out, nn), lambda b, k: (b, 0, 0))
        out_shape = jax.ShapeDtypeStruct((B, c_out, nn), F32)
    return pl.pallas_call(
        body,
        grid=(B, nk),
        in_specs=in_specs,
        out_specs=out_specs,
        out_shape=out_shape,
        scratch_shapes=[pltpu.VMEM((c_out, nn), F32)],
        compiler_params=pltpu.CompilerParams(
            dimension_semantics=("parallel", "arbitrary")),
        interpret=_INTERP,
    )(*args)


def _conv3_shift(xp, w9, bias, pos):
    def body(x_ref, w_ref, b_ref, pos_ref, o_ref):
        own = jnp.remainder(
            lax.broadcasted_iota(jnp.int32, (N_NODES, 1), 0), 56)
        acc = None
        for dy in range(3):
            for dx in range(3):
                start = dy * 56 + dx + 7
                sl = x_ref[0, start:start + N_NODES, :]
                z = _dot(sl, w_ref[dy * 3 + dx])
                if dx == 0:
                    z = jnp.where(own == 0, 0.0, z)
                elif dx == 2:
                    z = jnp.where(own == 55, 0.0, z)
                acc = z if acc is None else acc + z
        o_ref[...] = acc + b_ref[...] + pos_ref[...].T

    return pl.pallas_call(
        body,
        grid=(B,),
        in_specs=[
            pl.BlockSpec((1, N_NODES + 128, C), lambda b: (b, 0, 0)),
            pl.BlockSpec((9, C, C), lambda b: (0, 0, 0)),
            pl.BlockSpec((1, C), lambda b: (0, 0)),
            pl.BlockSpec((C, N_NODES), lambda b: (0, 0)),
        ],
        out_specs=pl.BlockSpec((N_NODES, C), lambda b: (b, 0)),
        out_shape=jax.ShapeDtypeStruct((B * N_NODES, C), F32),
        compiler_params=pltpu.CompilerParams(
            dimension_semantics=("parallel",)),
        interpret=_INTERP,
    )(xp, w9, bias.reshape(1, C), pos)


def _fc1_norm(x, w, bias):
    m = x.shape[0]

    def body(x_ref, w_ref, b_ref, y_ref, xn_ref, xnt_ref):
        y = _dot(x_ref[...], w_ref[...]) + b_ref[...]
        y_ref[...] = y
        n2 = jnp.sum(y * y, axis=1, keepdims=True)
        nrm = jnp.maximum(jnp.sqrt(n2), 1e-12)
        xn = y / nrm
        xn_ref[...] = xn
        xnt_ref[0] = xn.T

    return pl.pallas_call(
        body,
        grid=(B,),
        in_specs=[
            pl.BlockSpec((N_NODES, C), lambda i: (i, 0)),
            pl.BlockSpec((C, C), lambda i: (0, 0)),
            pl.BlockSpec((1, C), lambda i: (0, 0)),
        ],
        out_specs=[
            pl.BlockSpec((N_NODES, C), lambda i: (i, 0)),
            pl.BlockSpec((N_NODES, C), lambda i: (i, 0)),
            pl.BlockSpec((1, C, N_NODES), lambda i: (i, 0, 0)),
        ],
        out_shape=[
            jax.ShapeDtypeStruct((m, C), F32),
            jax.ShapeDtypeStruct((m, C), F32),
            jax.ShapeDtypeStruct((B, C, N_NODES), F32),
        ],
        compiler_params=pltpu.CompilerParams(
            dimension_semantics=("parallel",)),
        interpret=_INTERP,
    )(x, w, bias.reshape(1, C))


_KSCALE = np.float32(2.0 ** 27)
_I32MAX = np.int32(2**31 - 1)


def _topk_idx(xn, xnt, batch):
    nb = N_NODES // TM

    def body(xn_ref, xnt_ref, o_ref):
        x = xn_ref[...]
        xt = xnt_ref[0]
        sqr = jnp.sum(x * x, axis=1, keepdims=True)
        sqc = jnp.sum(xt * xt, axis=0, keepdims=True)
        ip = _dot(x, xt)
        d = jnp.maximum(sqr - 2.0 * ip + sqc, 0.0)
        ki = (d * _KSCALE).astype(jnp.int32)
        col = lax.broadcasted_iota(jnp.int32, (TM, N_NODES), 1)
        key = jnp.bitwise_or(jnp.bitwise_and(ki, jnp.int32(-4096)), col)
        cols = []
        mv = jnp.min(key, axis=1)
        cols.append(jnp.bitwise_and(mv, jnp.int32(4095)))
        for _ in range(KNN - 1):
            mv = jnp.min(jnp.where(key > mv[:, None], key, _I32MAX), axis=1)
            cols.append(jnp.bitwise_and(mv, jnp.int32(4095)))
        idx = jnp.stack(cols, axis=1) + batch * N_NODES
        pad = jnp.zeros((TM, 16 - KNN), jnp.int32)
        o_ref[...] = jnp.concatenate([idx, pad], axis=1)

    return pl.pallas_call(
        body,
        grid=(nb,),
        in_specs=[
            pl.BlockSpec((TM, C), lambda i: (batch * nb + i, 0)),
            pl.BlockSpec((1, C, N_NODES), lambda i: (batch, 0, 0)),
        ],
        out_specs=pl.BlockSpec((TM, 16), lambda i: (i, 0)),
        out_shape=jax.ShapeDtypeStruct((N_NODES, 16), jnp.int32),
        compiler_params=pltpu.CompilerParams(
            dimension_semantics=("parallel",)),
        interpret=_INTERP,
    )(xn, xnt)


_CHUNK_IDX = 72
_CHUNK_OUT = 8
_NW = 32


def _sc_gather_max(table, idxf, n_nodes):
    mesh = plsc.VectorSubcoreMesh(core_axis_name="c", subcore_axis_name="s")
    nv = C // 16
    _N_CHUNKS = n_nodes // _CHUNK_OUT
    _MAX_T = (_N_CHUNKS + _NW - 1) // _NW

    @functools.partial(
        pl.kernel,
        out_type=jax.ShapeDtypeStruct((n_nodes, C), F32),
        mesh=mesh,
        scratch_types=[
            pltpu.VMEM((_CHUNK_IDX,), jnp.int32),
            pltpu.VMEM((_CHUNK_IDX,), jnp.int32),
            pltpu.VMEM((_CHUNK_IDX, C), F32),
            pltpu.VMEM((_CHUNK_IDX, C), F32),
            pltpu.VMEM((_CHUNK_OUT, C), F32),
            pltpu.SemaphoreType.DMA,
            pltpu.SemaphoreType.DMA,
        ],
        compiler_params=pltpu.CompilerParams(use_tc_tiling_on_sc=False),
    )
    def k(tab_hbm, idx_hbm, out_hbm, idx0, idx1, rows0, rows1, out_v,
          sem0, sem1):
        wid = lax.axis_index("s") * 2 + lax.axis_index("c")
        idxb = [idx0, idx1]
        rowsb = [rows0, rows1]
        semb = [sem0, sem1]

        pltpu.sync_copy(idx_hbm.at[pl.ds(wid * _CHUNK_IDX, _CHUNK_IDX)], idx0)
        pltpu.make_async_copy(tab_hbm.at[idx0], rows0, sem0).start()

        @pl.loop(0, 2 * ((_MAX_T + 1) // 2), step=2)
        def _(tt):
            for j in range(2):
                t = tt + j
                c = wid + _NW * t

                @pl.when(c < _N_CHUNKS)
                def _():
                    pltpu.make_async_copy(
                        tab_hbm.at[idxb[j]], rowsb[j], semb[j]).wait()
                    cn = wid + _NW * (t + 1)

                    @pl.when(cn < _N_CHUNKS)
                    def _():
                        pltpu.sync_copy(
                            idx_hbm.at[pl.ds(cn * _CHUNK_IDX, _CHUNK_IDX)],
                            idxb[1 - j])
                        pltpu.make_async_copy(
                            tab_hbm.at[idxb[1 - j]], rowsb[1 - j],
                            semb[1 - j]).start()

                    @pl.loop(0, _CHUNK_OUT)
                    def _(nrow):
                        base = nrow * KNN
                        for v in range(nv):
                            sl = pl.ds(v * 16, 16)
                            acc = rowsb[j][base, sl]
                            for r in range(1, KNN):
                                acc = jnp.maximum(acc, rowsb[j][base + r, sl])
                            out_v[nrow, sl] = acc

                    pltpu.sync_copy(
                        out_v, out_hbm.at[pl.ds(c * _CHUNK_OUT, _CHUNK_OUT)])

    return k(table, idxf)


def _block_tail(y, g, x0, wa, wb, bmr, sg, beg, w2, b2, wf1, bf1, wf2, bf2):
    m = y.shape[0]

    def body(y_ref, g_ref, x0_ref, wa_ref, wb_ref, bmr_ref, sg_ref, beg_ref,
             w2_ref, b2_ref, wf1_ref, bf1_ref, wf2_ref, bf2_ref, o_ref):
        yv = y_ref[...]
        diff = g_ref[...] - yv
        z = _dot(yv, wa_ref[...]) + _dot(diff, wb_ref[...]) + bmr_ref[...]
        h = _gelu(z)
        h = _gelu(h * sg_ref[...] + beg_ref[...])
        xm = _dot(h, w2_ref[...]) + b2_ref[...] + x0_ref[...]
        tt = _gelu(_dot(xm, wf1_ref[...]) + bf1_ref[...])
        o_ref[...] = _dot(tt, wf2_ref[...]) + bf2_ref[...] + xm

    vec = lambda a: a.reshape(1, -1)
    row_spec = pl.BlockSpec((TM, C), lambda i: (i, 0))
    w_spec = pl.BlockSpec((C, C), lambda i: (0, 0))
    v_spec = pl.BlockSpec((1, C), lambda i: (0, 0))
    return pl.pallas_call(
        body,
        grid=(m // TM,),
        in_specs=[row_spec, row_spec, row_spec,
                  w_spec, w_spec, v_spec, v_spec, v_spec,
                  w_spec, v_spec, w_spec, v_spec, w_spec, v_spec],
        out_specs=row_spec,
        out_shape=jax.ShapeDtypeStruct((m, C), F32),
        compiler_params=pltpu.CompilerParams(
            dimension_semantics=("parallel",)),
        interpret=_INTERP,
    )(y, g, x0, wa, wb, vec(bmr), vec(sg), vec(beg),
      w2, vec(b2), wf1, vec(bf1), wf2, vec(bf2))


def _head(x, w1, b1, w2, b2):
    def body(x_ref, w1_ref, b1_ref, w2_ref, b2_ref, o_ref):
        xs = x_ref[...]
        mn = jnp.mean(xs.reshape(B, N_NODES, C), axis=1)
        z = _gelu(_dot(mn, w1_ref[...]) + b1_ref[...])
        o_ref[...] = _dot(z, w2_ref[...]) + b2_ref[...]

    n1 = w1.shape[1]
    n2 = w2.shape[1]
    return pl.pallas_call(
        body,
        in_specs=[
            pl.BlockSpec(x.shape, lambda: (0, 0)),
            pl.BlockSpec(w1.shape, lambda: (0, 0)),
            pl.BlockSpec((1, n1), lambda: (0, 0)),
            pl.BlockSpec(w2.shape, lambda: (0, 0)),
            pl.BlockSpec((1, n2), lambda: (0, 0)),
        ],
        out_specs=pl.BlockSpec((B, n2), lambda: (0, 0)),
        out_shape=jax.ShapeDtypeStruct((B, n2), F32),
        interpret=_INTERP,
    )(x, w1, b1.reshape(1, n1), w2, b2.reshape(1, n2))


def _fold(w2d, bias, g, be):
    s = g * _BN_S
    return w2d * s[None, :], bias * s + be


def _s2_weight_cm(w, g):
    o, i = w.shape[0], w.shape[1]
    ws = w * (g * _BN_S)[:, None, None, None]
    wp = jnp.zeros((2, 2, i, 2, 2, o), F32)
    for dy in range(3):
        cy, r = dy // 2, dy % 2
        for dx in range(3):
            cx, cc = dx // 2, dx % 2
            wp = wp.at[cy, cx, :, r, cc].set(ws[:, :, dy, dx].T)
    return wp


def _s2d_kernel(x):
    _, cch, hp, wp = x.shape
    h2, w2 = hp // 2, wp // 2

    def body(x_ref, o_ref):
        xv = x_ref[0]
        t1 = jnp.transpose(xv, (0, 2, 1))
        for c in (0, 1):
            tc = t1.reshape(cch, w2, 2, hp)[:, :, c, :]
            t2 = jnp.transpose(tc, (0, 2, 1))
            for r in (0, 1):
                o_ref[0, :, r, c] = t2.reshape(cch, h2, 2, w2)[:, :, r, :]

    return pl.pallas_call(
        body,
        grid=(B,),
        in_specs=[pl.BlockSpec((1, cch, hp, wp), lambda b: (b, 0, 0, 0))],
        out_specs=pl.BlockSpec((1, cch, 2, 2, h2, w2),
                               lambda b: (b, 0, 0, 0, 0, 0)),
        out_shape=jax.ShapeDtypeStruct((B, cch, 2, 2, h2, w2), F32),
        compiler_params=pltpu.CompilerParams(
            dimension_semantics=("parallel",)),
        interpret=_INTERP,
    )(x)


def kernel(inputs, params):
    p = params
    s = p['stem']

    xpad = jnp.pad(inputs, ((0, 0), (0, 0), (1, 1), (1, 1)))
    xs = _s2d_kernel(xpad).reshape(B, 12, 113, 113)
    p1 = jnp.concatenate(
        [xs[:, :, cy:cy + 112, cx:cx + 112] for cy in (0, 1)
         for cx in (0, 1)], axis=1).reshape(B, 48, 4 * N_NODES)
    w1 = _s2_weight_cm(s['W1'], s['g1']).reshape(48, 96).T.reshape(1, 96, 48)
    b1 = s['b1'] * (s['g1'] * _BN_S) + s['be1']
    y1 = _cmm(p1, w1, b1, act=True, nk=1, nn=4 * N_NODES, c_out=96)

    x1p = jnp.pad(y1.reshape(B, 96, 112, 112),
                  ((0, 0), (0, 0), (1, 1), (1, 1)))
    x1s = _s2d_kernel(x1p).reshape(B, 384, 57, 57)
    p2 = jnp.concatenate(
        [x1s[:, :, cy:cy + 56, cx:cx + 56] for cy in (0, 1)
         for cx in (0, 1)], axis=1).reshape(B, 1536, N_NODES)
    w2 = (_s2_weight_cm(s['W2'], s['g2'])
          .reshape(4, 384, 192).transpose(0, 2, 1))
    b2 = s['b2'] * (s['g2'] * _BN_S) + s['be2']
    y2 = _cmm(p2, w2, b2, act=True, nk=4, nn=N_NODES, c_out=C,
              transpose_out=True)

    s3 = s['g3'] * _BN_S
    w9 = (s['W3'].transpose(2, 3, 1, 0).reshape(9 * C, C) * s3[None, :]
          ).reshape(9, C, C)
    b3 = s['b3'] * s3 + s['be3']
    pos = p['pos_embed'].reshape(C, N_NODES)
    y2p = jnp.pad(y2.reshape(B, N_NODES, C), ((0, 0), (64, 64), (0, 0)))
    x0 = _conv3_shift(y2p, w9, b3, pos)

    for blk in p['blocks']:
        wf, bf = _fold(blk['fc1_W'][:, :, 0, 0].T, blk['fc1_b'],
                       blk['fc1_g'], blk['fc1_be'])
        y, xn, xnt = _fc1_norm(x0, wf, bf)
        gs = []
        for b in range(B):
            idx16 = _topk_idx(xn, xnt, b)
            idxf = idx16[:, :KNN].reshape(-1)
            gs.append(_sc_gather_max(y, idxf, N_NODES))
        gmax = jnp.concatenate(gs, axis=0)

        mr = blk['mr_W'][:, :, 0, 0]
        wa = mr[:, 0::2].T
        wb = mr[:, 1::2].T
        sg = blk['gbn_g'] * _BN_S
        beg = blk['gbn_be']
        w2e, b2e = _fold(blk['fc2_W'][:, :, 0, 0].T, blk['fc2_b'],
                         blk['fc2_g'], blk['fc2_be'])
        wf1, bf1 = _fold(blk['ffn1_W'][:, :, 0, 0].T, blk['ffn1_b'],
                         blk['ffn1_g'], blk['ffn1_be'])
        wf2, bf2 = _fold(blk['ffn2_W'][:, :, 0, 0].T, blk['ffn2_b'],
                         blk['ffn2_g'], blk['ffn2_be'])
        x0 = _block_tail(y, gmax, x0, wa, wb, blk['mr_b'], sg, beg,
                         w2e, b2e, wf1, bf1, wf2, bf2)

    h = p['head']
    wh1, bh1 = _fold(h['W1'][:, :, 0, 0].T, h['b1'], h['g1'], h['be1'])
    wh2 = h['W2'][:, :, 0, 0].T
    return _head(x0, wh1, bh1, wh2, h['b2'])
